# baseline (speedup 1.0000x reference)
import functools

import jax
import jax.numpy as jnp
from jax import lax
from jax.experimental import pallas as pl
from jax.experimental.pallas import tpu as pltpu
from jax.experimental.pallas import tpu_sc as plsc

T = 2048
DIN = 2048
D = 1024
F = 4096
E = 8
K = 2
B = 256
R = (T * K) // B + E
N = R * B
TB = 256
FB = 1024
NF = F // FB
EPAD = 128

_INV_SQRT2 = 0.7071067811865476


def _gelu(x):
    return 0.5 * x * (1.0 + jax.lax.erf(x * _INV_SQRT2))


def _gate_body(x_ref, wg1_ref, bg1_ref, wg2_ref, bg2_ref, meta_ref, xbf_ref):
    x = x_ref[...]
    xbf_ref[...] = x.astype(jnp.bfloat16)
    h = _gelu(jnp.dot(x, wg1_ref[...], preferred_element_type=jnp.float32)
              + bg1_ref[...])
    logits = jnp.dot(h, wg2_ref[...], preferred_element_type=jnp.float32) + bg2_ref[...]
    col = jax.lax.broadcasted_iota(jnp.int32, (TB, EPAD), 1)
    valid = col < E
    lm = jnp.where(valid, logits, jnp.float32(-1e30))
    m = jnp.max(lm, axis=1, keepdims=True)
    p = jnp.where(valid, jnp.exp(lm - m), 0.0)
    s = jnp.sum(p, axis=1, keepdims=True)
    probs = p / s
    m1 = jnp.max(probs, axis=1, keepdims=True)
    i1 = jnp.min(jnp.where(probs == m1, col, EPAD - 1), axis=1, keepdims=True)
    probs2 = jnp.where(col == i1, -1.0, probs)
    m2 = jnp.max(probs2, axis=1, keepdims=True)
    i2 = jnp.min(jnp.where(probs2 == m2, col, EPAD - 1), axis=1, keepdims=True)
    tot = m1 + m2
    w1 = m1 / tot
    w2 = m2 / tot
    col8 = jax.lax.broadcasted_iota(jnp.int32, (TB, 8), 1)
    out = (jnp.where(col8 == 0, i1.astype(jnp.float32), 0.0)
           + jnp.where(col8 == 1, i2.astype(jnp.float32), 0.0)
           + jnp.where(col8 == 2, w1, 0.0)
           + jnp.where(col8 == 3, w2, 0.0))
    meta_ref[...] = out


NC = 2
NS = 16
NW = NC * NS
GPW = N // NW
GCH = 48
TW = T // NW
CW = 32

_sc_mesh = plsc.VectorSubcoreMesh(core_axis_name="c", subcore_axis_name="s")


@functools.partial(
    pl.kernel, mesh=_sc_mesh,
    out_type=jax.ShapeDtypeStruct((N, DIN // 2), jnp.int32),
    scratch_types=[
        pltpu.VMEM((GCH,), jnp.int32),
        pltpu.VMEM((GCH,), jnp.int32),
        pltpu.VMEM((GCH, DIN // 2), jnp.int32),
        pltpu.VMEM((GCH, DIN // 2), jnp.int32),
        pltpu.SemaphoreType.DMA,
        pltpu.SemaphoreType.DMA,
    ],
)
def _sc_gather(idx_hbm, table_hbm, out_hbm, i0_v, i1_v, r0_v, r1_v, s0, s1):
    wid = lax.axis_index("s") * NC + lax.axis_index("c")
    base = wid * GPW
    nch = GPW // GCH
    idx_bufs = (i0_v, i1_v)
    row_bufs = (r0_v, r1_v)
    sems = (s0, s1)
    pltpu.sync_copy(idx_hbm.at[pl.ds(base, GCH)], i0_v)
    gathers = [pltpu.async_copy(table_hbm.at[i0_v], r0_v, s0)]
    for c in range(nch):
        b1 = (c + 1) % 2
        if c + 1 < nch:
            off1 = base + (c + 1) * GCH
            pltpu.sync_copy(idx_hbm.at[pl.ds(off1, GCH)], idx_bufs[b1])
            gathers.append(pltpu.async_copy(
                table_hbm.at[idx_bufs[b1]], row_bufs[b1], sems[b1]))
        gathers[c].wait()
        pltpu.sync_copy(row_bufs[c % 2], out_hbm.at[pl.ds(base + c * GCH, GCH)])


@functools.partial(
    pl.kernel, mesh=_sc_mesh,
    out_type=jax.ShapeDtypeStruct((T, D), jnp.float32),
    scratch_types=[
        pltpu.VMEM((CW,), jnp.int32),
        pltpu.VMEM((CW,), jnp.int32),
        pltpu.VMEM((CW, D), jnp.float32),
        pltpu.VMEM((CW, D), jnp.float32),
        pltpu.SemaphoreType.DMA,
    ],
)
def _sc_combine(p0_hbm, p1_hbm, yw_hbm, out_hbm, i0_v, i1_v, r0_v, r1_v, sem):
    wid = lax.axis_index("s") * NC + lax.axis_index("c")
    base = wid * TW
    for c in range(TW // CW):
        off = base + c * CW
        pltpu.sync_copy(p0_hbm.at[pl.ds(off, CW)], i0_v)
        pltpu.sync_copy(p1_hbm.at[pl.ds(off, CW)], i1_v)
        pltpu.async_copy(yw_hbm.at[i0_v], r0_v, sem).wait()
        pltpu.async_copy(yw_hbm.at[i1_v], r1_v, sem).wait()

        def _row(i, carry):
            for j in range(D // 16):
                sl = pl.ds(j * 16, 16)
                r0_v[i, sl] = r0_v[i, sl] + r1_v[i, sl]
            return carry

        lax.fori_loop(0, CW, _row, 0)
        pltpu.sync_copy(r0_v, out_hbm.at[pl.ds(off, CW)])


def _mm1_body(te_ref, tv_ref, xs_ref, w1_ref, b1_ref, h_ref):
    r = pl.program_id(1)

    @pl.when(tv_ref[r] == 1)
    def _():
        x32 = xs_ref[...].astype(jnp.float32)
        h_ref[...] = _gelu(
            jnp.dot(x32, w1_ref[0], preferred_element_type=jnp.float32)
            + b1_ref[0]).astype(jnp.bfloat16)

    @pl.when(tv_ref[r] == 0)
    def _():
        h_ref[...] = jnp.zeros_like(h_ref)


def _mm2_body(te_ref, tv_ref, h_ref, w2_ref, b2_ref, ws_ref, y_ref):
    r = pl.program_id(0)

    @pl.when(tv_ref[r] == 1)
    def _():
        h32 = h_ref[...].astype(jnp.float32)
        y_ref[...] = ws_ref[...] * jax.nn.sigmoid(
            jnp.dot(h32, w2_ref[0], preferred_element_type=jnp.float32)
            + b2_ref[0])

    @pl.when(tv_ref[r] == 0)
    def _():
        y_ref[...] = jnp.zeros_like(y_ref)


def kernel(combined, Wg1, bg1, Wg2, bg2, W1, b1, W2, b2):
    f32 = jnp.float32
    wg2p = jnp.pad(Wg2, ((0, 0), (0, EPAD - E)))
    bg2p = jnp.pad(bg2, (0, EPAD - E)).reshape(1, EPAD)
    bg1r = bg1.reshape(1, D)
    meta = pl.pallas_call(
        _gate_body,
        grid=(T // TB,),
        in_specs=[
            pl.BlockSpec((TB, DIN), lambda i: (i, 0)),
            pl.BlockSpec((DIN, D), lambda i: (0, 0)),
            pl.BlockSpec((1, D), lambda i: (0, 0)),
            pl.BlockSpec((D, EPAD), lambda i: (0, 0)),
            pl.BlockSpec((1, EPAD), lambda i: (0, 0)),
        ],
        out_specs=[pl.BlockSpec((TB, 8), lambda i: (i, 0)),
                   pl.BlockSpec((TB, DIN), lambda i: (i, 0))],
        out_shape=[jax.ShapeDtypeStruct((T, 8), f32),
                   jax.ShapeDtypeStruct((T, DIN), jnp.bfloat16)],
    )(combined, Wg1, bg1r, wg2p, bg2p)
    meta, xbf = meta

    e0 = meta[:, 0].astype(jnp.int32)
    e1 = meta[:, 1].astype(jnp.int32)
    w0 = meta[:, 2:3]
    w1v = meta[:, 3:4]

    e_all = jnp.stack([e0, e1], axis=1).reshape(-1)
    onehot = (e_all[:, None] == jnp.arange(E, dtype=jnp.int32)[None, :]).astype(jnp.int32)
    ranks = jnp.cumsum(onehot, axis=0)
    counts = ranks[-1]
    rank_p = jnp.take_along_axis(ranks, e_all[:, None], axis=1)[:, 0] - 1
    padded = ((counts + B - 1) // B) * B
    poff = jnp.concatenate([jnp.zeros(1, jnp.int32),
                            jnp.cumsum(padded)[:-1].astype(jnp.int32)])
    dst = poff[e_all] + rank_p
    tok = jnp.arange(T * K, dtype=jnp.int32) // K
    row_token = jnp.zeros(N, jnp.int32).at[dst].set(tok)
    pos = dst.reshape(T, K)
    tile_start = jnp.arange(R, dtype=jnp.int32) * B
    te = jnp.clip(jnp.searchsorted(poff, tile_start, side='right') - 1,
                  0, E - 1).astype(jnp.int32)
    tv = (tile_start < poff[te] + counts[te]).astype(jnp.int32)

    xpacked = jax.lax.bitcast_convert_type(
        xbf.reshape(T, DIN // 2, 2), jnp.int32)
    xsp = _sc_gather(row_token, xpacked)
    xs = jax.lax.bitcast_convert_type(xsp, jnp.bfloat16).reshape(N, DIN)

    w_all = jnp.stack([w0[:, 0], w1v[:, 0]], axis=1).reshape(-1)
    ws = jnp.zeros(N, f32).at[dst].set(w_all).reshape(N, 1)

    b1r = b1.reshape(E * NF, 1, FB)
    h = pl.pallas_call(
        _mm1_body,
        grid_spec=pltpu.PrefetchScalarGridSpec(
            num_scalar_prefetch=2,
            grid=(NF, R),
            in_specs=[
                pl.BlockSpec((B, DIN), lambda f, r, te, tv: (r, 0)),
                pl.BlockSpec((1, DIN, FB), lambda f, r, te, tv: (te[r], 0, f)),
                pl.BlockSpec((1, 1, FB), lambda f, r, te, tv: (te[r] * NF + f, 0, 0)),
            ],
            out_specs=pl.BlockSpec((B, FB), lambda f, r, te, tv: (r, f)),
        ),
        out_shape=jax.ShapeDtypeStruct((N, F), jnp.bfloat16),
    )(te, tv, xs, W1, b1r)

    b2r = b2.reshape(E, 1, D)
    y = pl.pallas_call(
        _mm2_body,
        grid_spec=pltpu.PrefetchScalarGridSpec(
            num_scalar_prefetch=2,
            grid=(R,),
            in_specs=[
                pl.BlockSpec((B, F), lambda r, te, tv: (r, 0)),
                pl.BlockSpec((1, F, D), lambda r, te, tv: (te[r], 0, 0)),
                pl.BlockSpec((1, 1, D), lambda r, te, tv: (te[r], 0, 0)),
                pl.BlockSpec((B, 1), lambda r, te, tv: (r, 0)),
            ],
            out_specs=pl.BlockSpec((B, D), lambda r, te, tv: (r, 0)),
        ),
        out_shape=jax.ShapeDtypeStruct((N, D), f32),
    )(te, tv, h, W2, b2r, ws)

    p0 = pos[:, 0]
    p1 = pos[:, 1]
    fused = _sc_combine(p0, p1, y)
    return fused


# speedup vs baseline: 1.7168x; 1.7168x over previous
import functools

import jax
import jax.numpy as jnp
from jax import lax
from jax.experimental import pallas as pl
from jax.experimental.pallas import tpu as pltpu
from jax.experimental.pallas import tpu_sc as plsc

T = 2048
DIN = 2048
D = 1024
F = 4096
E = 8
K = 2
B = 256
R = (T * K) // B + E
N = R * B
TB = 256
FB = 1024
NF = F // FB
EPAD = 128

_INV_SQRT2 = 0.7071067811865476


def _gelu(x):
    return 0.5 * x * (1.0 + jax.lax.erf(x * _INV_SQRT2))


def _gate_body(x_ref, wg1_ref, bg1_ref, wg2_ref, bg2_ref, meta_ref):
    x = x_ref[...]
    h = _gelu(jnp.dot(x, wg1_ref[...], preferred_element_type=jnp.float32)
              + bg1_ref[...])
    logits = jnp.dot(h, wg2_ref[...], preferred_element_type=jnp.float32) + bg2_ref[...]
    col = jax.lax.broadcasted_iota(jnp.int32, (TB, EPAD), 1)
    valid = col < E
    lm = jnp.where(valid, logits, jnp.float32(-1e30))
    m = jnp.max(lm, axis=1, keepdims=True)
    p = jnp.where(valid, jnp.exp(lm - m), 0.0)
    s = jnp.sum(p, axis=1, keepdims=True)
    probs = p / s
    m1 = jnp.max(probs, axis=1, keepdims=True)
    i1 = jnp.min(jnp.where(probs == m1, col, EPAD - 1), axis=1, keepdims=True)
    probs2 = jnp.where(col == i1, -1.0, probs)
    m2 = jnp.max(probs2, axis=1, keepdims=True)
    i2 = jnp.min(jnp.where(probs2 == m2, col, EPAD - 1), axis=1, keepdims=True)
    tot = m1 + m2
    w1 = m1 / tot
    w2 = m2 / tot
    col8 = jax.lax.broadcasted_iota(jnp.int32, (TB, 8), 1)
    out = (jnp.where(col8 == 0, i1.astype(jnp.float32), 0.0)
           + jnp.where(col8 == 1, i2.astype(jnp.float32), 0.0)
           + jnp.where(col8 == 2, w1, 0.0)
           + jnp.where(col8 == 3, w2, 0.0))
    meta_ref[...] = out


def _gather_body(rt_ref, x_ref, out_ref):
    rt = rt_ref[0]
    colt = jax.lax.broadcasted_iota(jnp.int32, (B, T), 1)
    p = (colt == rt).astype(jnp.float32)
    out_ref[...] = jnp.dot(
        p, x_ref[...], preferred_element_type=jnp.float32).astype(jnp.bfloat16)


NC = 2
NS = 16
NW = NC * NS
GPW = N // NW
GCH = 48
TW = T // NW
CW = 32

_sc_mesh = plsc.VectorSubcoreMesh(core_axis_name="c", subcore_axis_name="s")


@functools.partial(
    pl.kernel, mesh=_sc_mesh,
    out_type=jax.ShapeDtypeStruct((T, D), jnp.float32),
    scratch_types=[
        pltpu.VMEM((CW,), jnp.int32),
        pltpu.VMEM((CW,), jnp.int32),
        pltpu.VMEM((CW, D), jnp.float32),
        pltpu.VMEM((CW, D), jnp.float32),
        pltpu.SemaphoreType.DMA,
    ],
)
def _sc_combine(p0_hbm, p1_hbm, yw_hbm, out_hbm, i0_v, i1_v, r0_v, r1_v, sem):
    wid = lax.axis_index("s") * NC + lax.axis_index("c")
    base = wid * TW
    for c in range(TW // CW):
        off = base + c * CW
        pltpu.sync_copy(p0_hbm.at[pl.ds(off, CW)], i0_v)
        pltpu.sync_copy(p1_hbm.at[pl.ds(off, CW)], i1_v)
        pltpu.async_copy(yw_hbm.at[i0_v], r0_v, sem).wait()
        pltpu.async_copy(yw_hbm.at[i1_v], r1_v, sem).wait()

        def _row(i, carry):
            for j in range(D // 16):
                sl = pl.ds(j * 16, 16)
                r0_v[i, sl] = r0_v[i, sl] + r1_v[i, sl]
            return carry

        lax.fori_loop(0, CW, _row, 0)
        pltpu.sync_copy(r0_v, out_hbm.at[pl.ds(off, CW)])


def _mm1_body(te_ref, tv_ref, xs_ref, w1_ref, b1_ref, h_ref):
    r = pl.program_id(1)

    @pl.when(tv_ref[r] == 1)
    def _():
        x32 = xs_ref[...].astype(jnp.float32)
        h_ref[...] = _gelu(
            jnp.dot(x32, w1_ref[0], preferred_element_type=jnp.float32)
            + b1_ref[0]).astype(jnp.bfloat16)

    @pl.when(tv_ref[r] == 0)
    def _():
        h_ref[...] = jnp.zeros_like(h_ref)


def _mm2_body(te_ref, tv_ref, h_ref, w2_ref, b2_ref, ws_ref, y_ref):
    r = pl.program_id(0)

    @pl.when(tv_ref[r] == 1)
    def _():
        h32 = h_ref[...].astype(jnp.float32)
        y_ref[...] = ws_ref[...] * jax.nn.sigmoid(
            jnp.dot(h32, w2_ref[0], preferred_element_type=jnp.float32)
            + b2_ref[0])

    @pl.when(tv_ref[r] == 0)
    def _():
        y_ref[...] = jnp.zeros_like(y_ref)


def kernel(combined, Wg1, bg1, Wg2, bg2, W1, b1, W2, b2):
    f32 = jnp.float32
    wg2p = jnp.pad(Wg2, ((0, 0), (0, EPAD - E)))
    bg2p = jnp.pad(bg2, (0, EPAD - E)).reshape(1, EPAD)
    bg1r = bg1.reshape(1, D)
    meta = pl.pallas_call(
        _gate_body,
        grid=(T // TB,),
        in_specs=[
            pl.BlockSpec((TB, DIN), lambda i: (i, 0)),
            pl.BlockSpec((DIN, D), lambda i: (0, 0)),
            pl.BlockSpec((1, D), lambda i: (0, 0)),
            pl.BlockSpec((D, EPAD), lambda i: (0, 0)),
            pl.BlockSpec((1, EPAD), lambda i: (0, 0)),
        ],
        out_specs=pl.BlockSpec((TB, 8), lambda i: (i, 0)),
        out_shape=jax.ShapeDtypeStruct((T, 8), f32),
    )(combined, Wg1, bg1r, wg2p, bg2p)

    e0 = meta[:, 0].astype(jnp.int32)
    e1 = meta[:, 1].astype(jnp.int32)
    w0 = meta[:, 2:3]
    w1v = meta[:, 3:4]

    e_all = jnp.stack([e0, e1], axis=1).reshape(-1)
    onehot = (e_all[:, None] == jnp.arange(E, dtype=jnp.int32)[None, :]).astype(jnp.int32)
    ranks = jnp.cumsum(onehot, axis=0)
    counts = ranks[-1]
    rank_p = jnp.take_along_axis(ranks, e_all[:, None], axis=1)[:, 0] - 1
    padded = ((counts + B - 1) // B) * B
    poff = jnp.concatenate([jnp.zeros(1, jnp.int32),
                            jnp.cumsum(padded)[:-1].astype(jnp.int32)])
    dst = poff[e_all] + rank_p
    tok = jnp.arange(T * K, dtype=jnp.int32) // K
    row_token = jnp.zeros(N, jnp.int32).at[dst].set(tok)
    pos = dst.reshape(T, K)
    tile_start = jnp.arange(R, dtype=jnp.int32) * B
    te = jnp.clip(jnp.searchsorted(poff, tile_start, side='right') - 1,
                  0, E - 1).astype(jnp.int32)
    tv = (tile_start < poff[te] + counts[te]).astype(jnp.int32)

    xs = pl.pallas_call(
        _gather_body,
        grid=(R,),
        in_specs=[
            pl.BlockSpec((1, B, 1), lambda r: (r, 0, 0)),
            pl.BlockSpec((T, DIN), lambda r: (0, 0)),
        ],
        out_specs=pl.BlockSpec((B, DIN), lambda r: (r, 0)),
        out_shape=jax.ShapeDtypeStruct((N, DIN), jnp.bfloat16),
    )(row_token.reshape(R, B, 1), combined)

    w_all = jnp.stack([w0[:, 0], w1v[:, 0]], axis=1).reshape(-1)
    ws = jnp.zeros(N, f32).at[dst].set(w_all).reshape(N, 1)

    b1r = b1.reshape(E * NF, 1, FB)
    h = pl.pallas_call(
        _mm1_body,
        grid_spec=pltpu.PrefetchScalarGridSpec(
            num_scalar_prefetch=2,
            grid=(NF, R),
            in_specs=[
                pl.BlockSpec((B, DIN), lambda f, r, te, tv: (r, 0)),
                pl.BlockSpec((1, DIN, FB), lambda f, r, te, tv: (te[r], 0, f)),
                pl.BlockSpec((1, 1, FB), lambda f, r, te, tv: (te[r] * NF + f, 0, 0)),
            ],
            out_specs=pl.BlockSpec((B, FB), lambda f, r, te, tv: (r, f)),
        ),
        out_shape=jax.ShapeDtypeStruct((N, F), jnp.bfloat16),
    )(te, tv, xs, W1, b1r)

    b2r = b2.reshape(E, 1, D)
    y = pl.pallas_call(
        _mm2_body,
        grid_spec=pltpu.PrefetchScalarGridSpec(
            num_scalar_prefetch=2,
            grid=(R,),
            in_specs=[
                pl.BlockSpec((B, F), lambda r, te, tv: (r, 0)),
                pl.BlockSpec((1, F, D), lambda r, te, tv: (te[r], 0, 0)),
                pl.BlockSpec((1, 1, D), lambda r, te, tv: (te[r], 0, 0)),
                pl.BlockSpec((B, 1), lambda r, te, tv: (r, 0)),
            ],
            out_specs=pl.BlockSpec((B, D), lambda r, te, tv: (r, 0)),
        ),
        out_shape=jax.ShapeDtypeStruct((N, D), f32),
    )(te, tv, h, W2, b2r, ws)

    p0 = pos[:, 0]
    p1 = pos[:, 1]
    fused = _sc_combine(p0, p1, y)
    return fused


# speedup vs baseline: 1.7542x; 1.0217x over previous
import functools

import jax
import jax.numpy as jnp
from jax import lax
from jax.experimental import pallas as pl
from jax.experimental.pallas import tpu as pltpu
from jax.experimental.pallas import tpu_sc as plsc

T = 2048
DIN = 2048
D = 1024
F = 4096
E = 8
K = 2
B = 256
R = (T * K) // B + E
N = R * B
TB = 256
FB = 1024
NF = F // FB
EPAD = 128

_INV_SQRT2 = 0.7071067811865476


def _gelu(x):
    return 0.5 * x * (1.0 + jax.lax.erf(x * _INV_SQRT2))


def _gate_body(x_ref, wg1_ref, bg1_ref, wg2_ref, bg2_ref, meta_ref):
    x = x_ref[...]
    h = _gelu(jnp.dot(x, wg1_ref[...], preferred_element_type=jnp.float32)
              + bg1_ref[...])
    logits = jnp.dot(h, wg2_ref[...], preferred_element_type=jnp.float32) + bg2_ref[...]
    col = jax.lax.broadcasted_iota(jnp.int32, (TB, EPAD), 1)
    valid = col < E
    lm = jnp.where(valid, logits, jnp.float32(-1e30))
    m = jnp.max(lm, axis=1, keepdims=True)
    p = jnp.where(valid, jnp.exp(lm - m), 0.0)
    s = jnp.sum(p, axis=1, keepdims=True)
    probs = p / s
    m1 = jnp.max(probs, axis=1, keepdims=True)
    i1 = jnp.min(jnp.where(probs == m1, col, EPAD - 1), axis=1, keepdims=True)
    probs2 = jnp.where(col == i1, -1.0, probs)
    m2 = jnp.max(probs2, axis=1, keepdims=True)
    i2 = jnp.min(jnp.where(probs2 == m2, col, EPAD - 1), axis=1, keepdims=True)
    tot = m1 + m2
    w1 = m1 / tot
    w2 = m2 / tot
    col8 = jax.lax.broadcasted_iota(jnp.int32, (TB, 8), 1)
    out = (jnp.where(col8 == 0, i1.astype(jnp.float32), 0.0)
           + jnp.where(col8 == 1, i2.astype(jnp.float32), 0.0)
           + jnp.where(col8 == 2, w1, 0.0)
           + jnp.where(col8 == 3, w2, 0.0))
    meta_ref[...] = out


def _route_body(meta_ref, pos0_ref, pos1_ref, te_ref, tv_ref):
    meta = meta_ref[...]
    col8 = jax.lax.broadcasted_iota(jnp.int32, (T, 8), 1)
    e0 = meta[:, 0:1].astype(jnp.int32)
    e1 = meta[:, 1:2].astype(jnp.int32)
    oh0 = (col8 == e0).astype(jnp.float32)
    oh1 = (col8 == e1).astype(jnp.float32)
    oh01b = (oh0 + oh1).astype(jnp.bfloat16)
    ltri = (jax.lax.broadcasted_iota(jnp.int32, (T, T), 0)
            > jax.lax.broadcasted_iota(jnp.int32, (T, T), 1)).astype(jnp.bfloat16)
    cnt_before = jnp.dot(ltri, oh01b, preferred_element_type=jnp.float32)
    rank0 = jnp.sum(oh0 * cnt_before, axis=1, keepdims=True)
    rank1 = jnp.sum(oh1 * cnt_before, axis=1, keepdims=True)
    counts = cnt_before[T - 1:T, :] + oh0[T - 1:T, :] + oh1[T - 1:T, :]
    padded = (((counts.astype(jnp.int32) + (B - 1)) // B) * B).astype(jnp.float32)
    utri8 = (jax.lax.broadcasted_iota(jnp.int32, (8, 8), 0)
             < jax.lax.broadcasted_iota(jnp.int32, (8, 8), 1)).astype(jnp.float32)
    poff = jnp.dot(padded, utri8, preferred_element_type=jnp.float32)
    pos0_ref[...] = (rank0 + jnp.sum(oh0 * poff, 1, keepdims=True)).astype(jnp.int32)
    pos1_ref[...] = (rank1 + jnp.sum(oh1 * poff, 1, keepdims=True)).astype(jnp.int32)
    ts = (jax.lax.broadcasted_iota(jnp.int32, (R, 8), 0) * B).astype(jnp.float32)
    te = jnp.sum((poff <= ts).astype(jnp.int32), axis=1, keepdims=True) - 1
    ohte = (jax.lax.broadcasted_iota(jnp.int32, (R, 8), 1) == te).astype(jnp.float32)
    end_valid = jnp.sum(ohte * (poff + counts), axis=1, keepdims=True)
    te_ref[...] = te
    tv_ref[...] = (ts[:, 0:1] < end_valid).astype(jnp.int32)


def _gather_body(rt_ref, x_ref, out_ref):
    rt = rt_ref[0]
    colt = jax.lax.broadcasted_iota(jnp.int32, (B, T), 1)
    p = (colt == rt).astype(jnp.float32)
    out_ref[...] = jnp.dot(
        p, x_ref[...], preferred_element_type=jnp.float32).astype(jnp.bfloat16)


NC = 2
NS = 16
NW = NC * NS
GPW = N // NW
GCH = 48
TW = T // NW
CW = 32

_sc_mesh = plsc.VectorSubcoreMesh(core_axis_name="c", subcore_axis_name="s")


@functools.partial(
    pl.kernel, mesh=_sc_mesh,
    out_type=jax.ShapeDtypeStruct((T, D), jnp.float32),
    scratch_types=[
        pltpu.VMEM((CW,), jnp.int32),
        pltpu.VMEM((CW,), jnp.int32),
        pltpu.VMEM((CW, D), jnp.float32),
        pltpu.VMEM((CW, D), jnp.float32),
        pltpu.SemaphoreType.DMA,
    ],
)
def _sc_combine(p0_hbm, p1_hbm, yw_hbm, out_hbm, i0_v, i1_v, r0_v, r1_v, sem):
    wid = lax.axis_index("s") * NC + lax.axis_index("c")
    base = wid * TW
    for c in range(TW // CW):
        off = base + c * CW
        pltpu.sync_copy(p0_hbm.at[pl.ds(off, CW)], i0_v)
        pltpu.sync_copy(p1_hbm.at[pl.ds(off, CW)], i1_v)
        pltpu.async_copy(yw_hbm.at[i0_v], r0_v, sem).wait()
        pltpu.async_copy(yw_hbm.at[i1_v], r1_v, sem).wait()

        def _row(i, carry):
            for j in range(D // 16):
                sl = pl.ds(j * 16, 16)
                r0_v[i, sl] = r0_v[i, sl] + r1_v[i, sl]
            return carry

        lax.fori_loop(0, CW, _row, 0)
        pltpu.sync_copy(r0_v, out_hbm.at[pl.ds(off, CW)])


def _mm1_body(te_ref, tv_ref, xs_ref, w1_ref, b1_ref, h_ref):
    r = pl.program_id(1)

    @pl.when(tv_ref[r] == 1)
    def _():
        x32 = xs_ref[...].astype(jnp.float32)
        h_ref[...] = _gelu(
            jnp.dot(x32, w1_ref[0], preferred_element_type=jnp.float32)
            + b1_ref[0]).astype(jnp.bfloat16)

    @pl.when(tv_ref[r] == 0)
    def _():
        h_ref[...] = jnp.zeros_like(h_ref)


def _mm2_body(te_ref, tv_ref, h_ref, w2_ref, b2_ref, ws_ref, y_ref):
    r = pl.program_id(0)

    @pl.when(tv_ref[r] == 1)
    def _():
        h32 = h_ref[...].astype(jnp.float32)
        y_ref[...] = ws_ref[...] * jax.nn.sigmoid(
            jnp.dot(h32, w2_ref[0], preferred_element_type=jnp.float32)
            + b2_ref[0])

    @pl.when(tv_ref[r] == 0)
    def _():
        y_ref[...] = jnp.zeros_like(y_ref)


def kernel(combined, Wg1, bg1, Wg2, bg2, W1, b1, W2, b2):
    f32 = jnp.float32
    wg2p = jnp.pad(Wg2, ((0, 0), (0, EPAD - E)))
    bg2p = jnp.pad(bg2, (0, EPAD - E)).reshape(1, EPAD)
    bg1r = bg1.reshape(1, D)
    meta = pl.pallas_call(
        _gate_body,
        grid=(T // TB,),
        in_specs=[
            pl.BlockSpec((TB, DIN), lambda i: (i, 0)),
            pl.BlockSpec((DIN, D), lambda i: (0, 0)),
            pl.BlockSpec((1, D), lambda i: (0, 0)),
            pl.BlockSpec((D, EPAD), lambda i: (0, 0)),
            pl.BlockSpec((1, EPAD), lambda i: (0, 0)),
        ],
        out_specs=pl.BlockSpec((TB, 8), lambda i: (i, 0)),
        out_shape=jax.ShapeDtypeStruct((T, 8), f32),
    )(combined, Wg1, bg1r, wg2p, bg2p)

    pos0q, pos1q, teq, tvq = pl.pallas_call(
        _route_body,
        in_specs=[pl.BlockSpec((T, 8), lambda: (0, 0))],
        out_specs=[
            pl.BlockSpec((T, 1), lambda: (0, 0)),
            pl.BlockSpec((T, 1), lambda: (0, 0)),
            pl.BlockSpec((R, 1), lambda: (0, 0)),
            pl.BlockSpec((R, 1), lambda: (0, 0)),
        ],
        out_shape=[
            jax.ShapeDtypeStruct((T, 1), jnp.int32),
            jax.ShapeDtypeStruct((T, 1), jnp.int32),
            jax.ShapeDtypeStruct((R, 1), jnp.int32),
            jax.ShapeDtypeStruct((R, 1), jnp.int32),
        ],
    )(meta)
    p0 = pos0q[:, 0]
    p1 = pos1q[:, 0]
    te = teq[:, 0]
    tv = tvq[:, 0]
    tok = jnp.arange(T, dtype=jnp.int32)
    row_token = jnp.zeros(N, jnp.int32).at[p0].set(tok).at[p1].set(tok)
    ws = (jnp.zeros(N, f32).at[p0].set(meta[:, 2])
          .at[p1].set(meta[:, 3]).reshape(N, 1))

    xs = pl.pallas_call(
        _gather_body,
        grid=(R,),
        in_specs=[
            pl.BlockSpec((1, B, 1), lambda r: (r, 0, 0)),
            pl.BlockSpec((T, DIN), lambda r: (0, 0)),
        ],
        out_specs=pl.BlockSpec((B, DIN), lambda r: (r, 0)),
        out_shape=jax.ShapeDtypeStruct((N, DIN), jnp.bfloat16),
    )(row_token.reshape(R, B, 1), combined)

    b1r = b1.reshape(E * NF, 1, FB)
    h = pl.pallas_call(
        _mm1_body,
        grid_spec=pltpu.PrefetchScalarGridSpec(
            num_scalar_prefetch=2,
            grid=(NF, R),
            in_specs=[
                pl.BlockSpec((B, DIN), lambda f, r, te, tv: (r, 0)),
                pl.BlockSpec((1, DIN, FB), lambda f, r, te, tv: (te[r], 0, f)),
                pl.BlockSpec((1, 1, FB), lambda f, r, te, tv: (te[r] * NF + f, 0, 0)),
            ],
            out_specs=pl.BlockSpec((B, FB), lambda f, r, te, tv: (r, f)),
        ),
        out_shape=jax.ShapeDtypeStruct((N, F), jnp.bfloat16),
    )(te, tv, xs, W1, b1r)

    b2r = b2.reshape(E, 1, D)
    y = pl.pallas_call(
        _mm2_body,
        grid_spec=pltpu.PrefetchScalarGridSpec(
            num_scalar_prefetch=2,
            grid=(R,),
            in_specs=[
                pl.BlockSpec((B, F), lambda r, te, tv: (r, 0)),
                pl.BlockSpec((1, F, D), lambda r, te, tv: (te[r], 0, 0)),
                pl.BlockSpec((1, 1, D), lambda r, te, tv: (te[r], 0, 0)),
                pl.BlockSpec((B, 1), lambda r, te, tv: (r, 0)),
            ],
            out_specs=pl.BlockSpec((B, D), lambda r, te, tv: (r, 0)),
        ),
        out_shape=jax.ShapeDtypeStruct((N, D), f32),
    )(te, tv, h, W2, b2r, ws)

    fused = _sc_combine(p0, p1, y)
    return fused


# speedup vs baseline: 1.7717x; 1.0100x over previous
import functools

import jax
import jax.numpy as jnp
from jax import lax
from jax.experimental import pallas as pl
from jax.experimental.pallas import tpu as pltpu
from jax.experimental.pallas import tpu_sc as plsc

T = 2048
DIN = 2048
D = 1024
F = 4096
E = 8
K = 2
B = 256
R = (T * K) // B + E
N = R * B
TB = 256
FB = 1024
NF = F // FB
EPAD = 128

_INV_SQRT2 = 0.7071067811865476


def _gelu(x):
    return 0.5 * x * (1.0 + jax.lax.erf(x * _INV_SQRT2))


def _gate_body(x_ref, wg1_ref, bg1_ref, wg2_ref, bg2_ref, meta_ref):
    x = x_ref[...]
    h = _gelu(jnp.dot(x, wg1_ref[...], preferred_element_type=jnp.float32)
              + bg1_ref[...])
    logits = jnp.dot(h, wg2_ref[...], preferred_element_type=jnp.float32) + bg2_ref[...]
    col = jax.lax.broadcasted_iota(jnp.int32, (TB, EPAD), 1)
    valid = col < E
    lm = jnp.where(valid, logits, jnp.float32(-1e30))
    m = jnp.max(lm, axis=1, keepdims=True)
    p = jnp.where(valid, jnp.exp(lm - m), 0.0)
    s = jnp.sum(p, axis=1, keepdims=True)
    probs = p / s
    m1 = jnp.max(probs, axis=1, keepdims=True)
    i1 = jnp.min(jnp.where(probs == m1, col, EPAD - 1), axis=1, keepdims=True)
    probs2 = jnp.where(col == i1, -1.0, probs)
    m2 = jnp.max(probs2, axis=1, keepdims=True)
    i2 = jnp.min(jnp.where(probs2 == m2, col, EPAD - 1), axis=1, keepdims=True)
    tot = m1 + m2
    w1 = m1 / tot
    w2 = m2 / tot
    col8 = jax.lax.broadcasted_iota(jnp.int32, (TB, 8), 1)
    out = (jnp.where(col8 == 0, i1.astype(jnp.float32), 0.0)
           + jnp.where(col8 == 1, i2.astype(jnp.float32), 0.0)
           + jnp.where(col8 == 2, w1, 0.0)
           + jnp.where(col8 == 3, w2, 0.0))
    meta_ref[...] = out


def _route_body(meta_ref, pos0_ref, pos1_ref, te_ref, tv_ref):
    meta = meta_ref[...]
    col8 = jax.lax.broadcasted_iota(jnp.int32, (T, 8), 1)
    e0 = meta[:, 0:1].astype(jnp.int32)
    e1 = meta[:, 1:2].astype(jnp.int32)
    oh0 = (col8 == e0).astype(jnp.float32)
    oh1 = (col8 == e1).astype(jnp.float32)
    oh01b = (oh0 + oh1).astype(jnp.bfloat16)
    ltri = (jax.lax.broadcasted_iota(jnp.int32, (T, T), 0)
            > jax.lax.broadcasted_iota(jnp.int32, (T, T), 1)).astype(jnp.bfloat16)
    cnt_before = jnp.dot(ltri, oh01b, preferred_element_type=jnp.float32)
    rank0 = jnp.sum(oh0 * cnt_before, axis=1, keepdims=True)
    rank1 = jnp.sum(oh1 * cnt_before, axis=1, keepdims=True)
    counts = cnt_before[T - 1:T, :] + oh0[T - 1:T, :] + oh1[T - 1:T, :]
    padded = (((counts.astype(jnp.int32) + (B - 1)) // B) * B).astype(jnp.float32)
    utri8 = (jax.lax.broadcasted_iota(jnp.int32, (8, 8), 0)
             < jax.lax.broadcasted_iota(jnp.int32, (8, 8), 1)).astype(jnp.float32)
    poff = jnp.dot(padded, utri8, preferred_element_type=jnp.float32)
    pos0_ref[...] = (rank0 + jnp.sum(oh0 * poff, 1, keepdims=True)).astype(jnp.int32)
    pos1_ref[...] = (rank1 + jnp.sum(oh1 * poff, 1, keepdims=True)).astype(jnp.int32)
    ts = (jax.lax.broadcasted_iota(jnp.int32, (R, 8), 0) * B).astype(jnp.float32)
    te = jnp.sum((poff <= ts).astype(jnp.int32), axis=1, keepdims=True) - 1
    ohte = (jax.lax.broadcasted_iota(jnp.int32, (R, 8), 1) == te).astype(jnp.float32)
    end_valid = jnp.sum(ohte * (poff + counts), axis=1, keepdims=True)
    te_ref[...] = te
    tv_ref[...] = (ts[:, 0:1] < end_valid).astype(jnp.int32)


def _gather_body(tv_ref, rt_ref, x_ref, out_ref):
    r = pl.program_id(0)

    @pl.when(tv_ref[r] == 1)
    def _():
        rt = rt_ref[0]
        colt = jax.lax.broadcasted_iota(jnp.int32, (B, T), 1)
        p = (colt == rt).astype(jnp.float32)
        out_ref[...] = jnp.dot(
            p, x_ref[...], preferred_element_type=jnp.float32).astype(jnp.bfloat16)


NC = 2
NS = 16
NW = NC * NS
GPW = N // NW
GCH = 48
TW = T // NW
CW = 32

_sc_mesh = plsc.VectorSubcoreMesh(core_axis_name="c", subcore_axis_name="s")


@functools.partial(
    pl.kernel, mesh=_sc_mesh,
    out_type=jax.ShapeDtypeStruct((T, D), jnp.float32),
    scratch_types=[
        pltpu.VMEM((CW,), jnp.int32),
        pltpu.VMEM((CW,), jnp.int32),
        pltpu.VMEM((CW, D), jnp.float32),
        pltpu.VMEM((CW, D), jnp.float32),
        pltpu.SemaphoreType.DMA,
    ],
)
def _sc_combine(p0_hbm, p1_hbm, yw_hbm, out_hbm, i0_v, i1_v, r0_v, r1_v, sem):
    wid = lax.axis_index("s") * NC + lax.axis_index("c")
    base = wid * TW
    for c in range(TW // CW):
        off = base + c * CW
        pltpu.sync_copy(p0_hbm.at[pl.ds(off, CW)], i0_v)
        pltpu.sync_copy(p1_hbm.at[pl.ds(off, CW)], i1_v)
        pltpu.async_copy(yw_hbm.at[i0_v], r0_v, sem).wait()
        pltpu.async_copy(yw_hbm.at[i1_v], r1_v, sem).wait()

        def _row(i, carry):
            for j in range(D // 16):
                sl = pl.ds(j * 16, 16)
                r0_v[i, sl] = r0_v[i, sl] + r1_v[i, sl]
            return carry

        lax.fori_loop(0, CW, _row, 0)
        pltpu.sync_copy(r0_v, out_hbm.at[pl.ds(off, CW)])


def _mm1_body(te_ref, tv_ref, xs_ref, w1_ref, b1_ref, h_ref):
    r = pl.program_id(1)

    @pl.when(tv_ref[r] == 1)
    def _():
        x32 = xs_ref[...].astype(jnp.float32)
        h_ref[...] = _gelu(
            jnp.dot(x32, w1_ref[0], preferred_element_type=jnp.float32)
            + b1_ref[0]).astype(jnp.bfloat16)

    @pl.when(tv_ref[r] == 0)
    def _():
        h_ref[...] = jnp.zeros_like(h_ref)


def _mm2_body(te_ref, tv_ref, h_ref, w2_ref, b2_ref, ws_ref, y_ref):
    r = pl.program_id(0)

    @pl.when(tv_ref[r] == 1)
    def _():
        h32 = h_ref[...].astype(jnp.float32)
        y_ref[...] = ws_ref[...] * jax.nn.sigmoid(
            jnp.dot(h32, w2_ref[0], preferred_element_type=jnp.float32)
            + b2_ref[0])

    @pl.when(tv_ref[r] == 0)
    def _():
        y_ref[...] = jnp.zeros_like(y_ref)


def kernel(combined, Wg1, bg1, Wg2, bg2, W1, b1, W2, b2):
    f32 = jnp.float32
    wg2p = jnp.pad(Wg2, ((0, 0), (0, EPAD - E)))
    bg2p = jnp.pad(bg2, (0, EPAD - E)).reshape(1, EPAD)
    bg1r = bg1.reshape(1, D)
    meta = pl.pallas_call(
        _gate_body,
        grid=(T // TB,),
        in_specs=[
            pl.BlockSpec((TB, DIN), lambda i: (i, 0)),
            pl.BlockSpec((DIN, D), lambda i: (0, 0)),
            pl.BlockSpec((1, D), lambda i: (0, 0)),
            pl.BlockSpec((D, EPAD), lambda i: (0, 0)),
            pl.BlockSpec((1, EPAD), lambda i: (0, 0)),
        ],
        out_specs=pl.BlockSpec((TB, 8), lambda i: (i, 0)),
        out_shape=jax.ShapeDtypeStruct((T, 8), f32),
    )(combined, Wg1, bg1r, wg2p, bg2p)

    pos0q, pos1q, teq, tvq = pl.pallas_call(
        _route_body,
        in_specs=[pl.BlockSpec((T, 8), lambda: (0, 0))],
        out_specs=[
            pl.BlockSpec((T, 1), lambda: (0, 0)),
            pl.BlockSpec((T, 1), lambda: (0, 0)),
            pl.BlockSpec((R, 1), lambda: (0, 0)),
            pl.BlockSpec((R, 1), lambda: (0, 0)),
        ],
        out_shape=[
            jax.ShapeDtypeStruct((T, 1), jnp.int32),
            jax.ShapeDtypeStruct((T, 1), jnp.int32),
            jax.ShapeDtypeStruct((R, 1), jnp.int32),
            jax.ShapeDtypeStruct((R, 1), jnp.int32),
        ],
    )(meta)
    p0 = pos0q[:, 0]
    p1 = pos1q[:, 0]
    te = teq[:, 0]
    tv = tvq[:, 0]
    tok = jnp.arange(T, dtype=jnp.int32)
    row_token = jnp.zeros(N, jnp.int32).at[p0].set(tok).at[p1].set(tok)
    ws = (jnp.zeros(N, f32).at[p0].set(meta[:, 2])
          .at[p1].set(meta[:, 3]).reshape(N, 1))

    xs = pl.pallas_call(
        _gather_body,
        grid_spec=pltpu.PrefetchScalarGridSpec(
            num_scalar_prefetch=1,
            grid=(R,),
            in_specs=[
                pl.BlockSpec((1, B, 1), lambda r, tv: (r, 0, 0)),
                pl.BlockSpec((T, DIN), lambda r, tv: (0, 0)),
            ],
            out_specs=pl.BlockSpec((B, DIN), lambda r, tv: (r, 0)),
        ),
        out_shape=jax.ShapeDtypeStruct((N, DIN), jnp.bfloat16),
    )(tv, row_token.reshape(R, B, 1), combined)

    b1r = b1.reshape(E * NF, 1, FB)
    h = pl.pallas_call(
        _mm1_body,
        grid_spec=pltpu.PrefetchScalarGridSpec(
            num_scalar_prefetch=2,
            grid=(NF, R),
            in_specs=[
                pl.BlockSpec((B, DIN), lambda f, r, te, tv: (r, 0)),
                pl.BlockSpec((1, DIN, FB), lambda f, r, te, tv: (te[r], 0, f)),
                pl.BlockSpec((1, 1, FB), lambda f, r, te, tv: (te[r] * NF + f, 0, 0)),
            ],
            out_specs=pl.BlockSpec((B, FB), lambda f, r, te, tv: (r, f)),
        ),
        out_shape=jax.ShapeDtypeStruct((N, F), jnp.bfloat16),
    )(te, tv, xs, W1, b1r)

    b2r = b2.reshape(E, 1, D)
    y = pl.pallas_call(
        _mm2_body,
        grid_spec=pltpu.PrefetchScalarGridSpec(
            num_scalar_prefetch=2,
            grid=(R,),
            in_specs=[
                pl.BlockSpec((B, F), lambda r, te, tv: (r, 0)),
                pl.BlockSpec((1, F, D), lambda r, te, tv: (te[r], 0, 0)),
                pl.BlockSpec((1, 1, D), lambda r, te, tv: (te[r], 0, 0)),
                pl.BlockSpec((B, 1), lambda r, te, tv: (r, 0)),
            ],
            out_specs=pl.BlockSpec((B, D), lambda r, te, tv: (r, 0)),
        ),
        out_shape=jax.ShapeDtypeStruct((N, D), f32),
    )(te, tv, h, W2, b2r, ws)

    fused = _sc_combine(p0, p1, y)
    return fused
